# baseline (speedup 1.0000x reference)
import functools

import jax
import jax.numpy as jnp
from jax import lax
from jax.experimental import pallas as pl
from jax.experimental.pallas import tpu as pltpu
from jax.experimental.pallas import tpu_sc as plsc

N = 100000
E = 1600000
F = 32
DC = 128

NC = 2
NS = 16

RN = N // NC
DUMP0 = 50048
NDUMP = 64
RPAD = 50176
ZROWS = 784

K = 160
NCH = 625
EPT = NCH * K


def _sc_scatter(h, eflat, compute_w):
  mesh = plsc.VectorSubcoreMesh(
      core_axis_name="c", subcore_axis_name="s", num_cores=NC,
      num_subcores=NS)
  if compute_w:
    out_type = [jax.ShapeDtypeStruct((N, F), jnp.float32),
                jax.ShapeDtypeStruct((N,), jnp.float32)]
  else:
    out_type = jax.ShapeDtypeStruct((N, F), jnp.float32)

  scratch = (
      [pltpu.VMEM((K,), jnp.int32) for _ in range(4)]
      + [pltpu.VMEM((K,), jnp.int32) for _ in range(4)]
      + [pltpu.VMEM((K,), jnp.int32) for _ in range(4)]
      + [pltpu.VMEM((K, F), jnp.float32) for _ in range(4)]
      + [pltpu.VMEM((K,), jnp.float32)]
      + ([pltpu.VMEM((3136,), jnp.float32)] if compute_w else [])
      + [pltpu.VMEM_SHARED((RPAD, F), jnp.float32)]
      + ([pltpu.VMEM_SHARED((RPAD,), jnp.float32)] if compute_w else [])
      + [pltpu.SemaphoreType.DMA] * 12
  )

  def body(h_hbm, e_hbm, *refs):
    if compute_w:
      agg_out, w_out = refs[0], refs[1]
      refs = refs[2:]
    else:
      agg_out = refs[0]
      refs = refs[1:]
    sidx = refs[0:4]
    didx = refs[4:8]
    lix = refs[8:12]
    rws = refs[12:16]
    onesb = refs[16]
    if compute_w:
      zb1 = refs[17]
      agg_sh, w_sh = refs[18], refs[19]
      sems = refs[20:]
    else:
      agg_sh = refs[17]
      w_sh = None
      sems = refs[18:]
    isem = sems[0:4]
    gsem = sems[4:8]
    ssem = sems[8:12]

    c = lax.axis_index("c")
    s = lax.axis_index("s")
    b0 = c * RN

    zv = jnp.zeros((16,), jnp.float32)

    def zfill(i, _):
      rws[0][i, pl.ds(0, 16)] = zv
      rws[0][i, pl.ds(16, 16)] = zv
      return _
    lax.fori_loop(0, K, zfill, None)

    ov = jnp.ones((16,), jnp.float32)
    for j in range(K // 16):
      onesb[pl.ds(j * 16, 16)] = ov

    for q in range(3136 // K):
      pltpu.sync_copy(rws[0], agg_sh.at[pl.ds(s * 3136 + q * K, K)])
    zrem = 3136 % K
    if zrem:
      pltpu.sync_copy(rws[0].at[pl.ds(0, zrem)],
                      agg_sh.at[pl.ds(s * 3136 + 3136 - zrem, zrem)])
    if compute_w:
      def zfill1(i, _):
        zb1[pl.ds(i * 16, 16)] = zv
        return _
      lax.fori_loop(0, 3136 // 16, zfill1, None)
      pltpu.sync_copy(zb1, w_sh.at[pl.ds(s * 3136, 3136)])
    plsc.subcore_barrier()

    def eslice(t, which):
      off = pl.multiple_of(which * E + (s * NCH + t) * K, K)
      return e_hbm.at[pl.ds(off, K)]

    def idload_fire(t, x):
      pltpu.async_copy(eslice(t, 0), sidx[x], isem[x])
      pltpu.async_copy(eslice(t, 1), didx[x], isem[x])

    def idload_wait(t, x):
      pltpu.make_async_copy(eslice(t, 0), sidx[x], isem[x]).wait()
      pltpu.make_async_copy(eslice(t, 1), didx[x], isem[x]).wait()

    def gather_fire(x):
      pltpu.async_copy(h_hbm.at[sidx[x]], rws[x], gsem[x])

    def gather_drain(x):
      pltpu.make_async_copy(h_hbm.at[sidx[x]], rws[x], gsem[x]).wait()

    def scatter_fire(x):
      pltpu.async_copy(rws[x], agg_sh.at[lix[x]], ssem[x], add=True)
      if compute_w:
        pltpu.async_copy(onesb, w_sh.at[lix[x]], ssem[x], add=True)

    def scatter_drain(x):
      pltpu.make_async_copy(rws[x], agg_sh.at[lix[x]], ssem[x]).wait()
      if compute_w:
        pltpu.make_async_copy(onesb, w_sh.at[lix[x]], ssem[x]).wait()

    def mkidx(x):
      for ii in range(K // 16):
        d = didx[x][pl.ds(ii * 16, 16)]
        inr = (d >= b0) & (d < b0 + RN)
        loc = d - b0
        dmp = DUMP0 + (d & (NDUMP - 1))
        lix[x][pl.ds(ii * 16, 16)] = jnp.where(inr, loc, dmp)

    idload_fire(0, 0)
    idload_fire(1, 1)

    def iter_body(i, _):
      t0 = 4 * i
      for j in range(4):
        t = t0 + j
        x = j
        o = (j + 2) % 4

        @pl.when(t >= 4)
        def _():
          scatter_drain(x)

        idload_wait(t, x)
        gather_fire(x)
        mkidx(x)

        @pl.when(t >= 2)
        def _():
          gather_drain(o)
          scatter_fire(o)

        @pl.when(t < NCH - 2)
        def _():
          idload_fire(t + 2, o)
      return _
    lax.fori_loop(0, NCH // 4, iter_body, None)

    scatter_drain(0)
    idload_wait(NCH - 1, 0)
    gather_fire(0)
    mkidx(0)
    gather_drain(2)
    scatter_fire(2)
    gather_drain(3)
    scatter_fire(3)
    gather_drain(0)
    scatter_fire(0)
    scatter_drain(1)
    scatter_drain(2)
    scatter_drain(3)
    scatter_drain(0)
    plsc.subcore_barrier()

    obase = c * RN

    @pl.when(s < NS - 1)
    def _():
      pltpu.sync_copy(agg_sh.at[pl.ds(s * 3128, 3128)],
                      agg_out.at[pl.ds(obase + s * 3128, 3128)])
      if compute_w:
        pltpu.sync_copy(w_sh.at[pl.ds(s * 3128, 3128)],
                        w_out.at[pl.ds(obase + s * 3128, 3128)])

    @pl.when(s == NS - 1)
    def _():
      pltpu.sync_copy(agg_sh.at[pl.ds(46920, 3080)],
                      agg_out.at[pl.ds(obase + 46920, 3080)])
      if compute_w:
        pltpu.sync_copy(w_sh.at[pl.ds(46920, 3080)],
                        w_out.at[pl.ds(obase + 46920, 3080)])

  fn = pl.kernel(
      body, out_type=out_type, mesh=mesh, scratch_types=scratch,
      compiler_params=pltpu.CompilerParams(use_tc_tiling_on_sc=False))
  return fn(h, eflat)


def _lrelu(x):
  return jnp.where(x >= 0, x, 0.1 * x)


def _init_body(cont_ref, pw_ref, pb_ref, out_ref):
  x = jnp.dot(cont_ref[...], pw_ref[...],
              preferred_element_type=jnp.float32) + pb_ref[...]
  out_ref[...] = _lrelu(x)


def _tc_init(content, proj_W, proj_b):
  blk = 10000
  return pl.pallas_call(
      _init_body,
      grid=(N // blk,),
      in_specs=[
          pl.BlockSpec((blk, DC), lambda i: (i, 0)),
          pl.BlockSpec((DC, F), lambda i: (0, 0)),
          pl.BlockSpec((1, F), lambda i: (0, 0)),
      ],
      out_specs=pl.BlockSpec((blk, F), lambda i: (i, 0)),
      out_shape=jax.ShapeDtypeStruct((N, F), jnp.float32),
  )(content, proj_W, proj_b.reshape(1, F))


def _make_comb_body(prediction_layer):
  def body(h_ref, agg_ref, w1_ref, b1_ref, w2_ref, b2_ref, out_ref):
    h = h_ref[...]
    a = agg_ref[...]
    hc = jnp.concatenate([h, a], axis=1)
    z = jnp.dot(hc, w1_ref[...], preferred_element_type=jnp.float32)
    z = _lrelu(z + b1_ref[...])
    z = jnp.dot(z, w2_ref[...], preferred_element_type=jnp.float32)
    z = z + b2_ref[...]
    if not prediction_layer:
      z = _lrelu(z)
    nrm = jnp.sqrt(jnp.sum(z * z, axis=1, keepdims=True))
    out_ref[...] = z / jnp.maximum(nrm, 1e-6)
  return body


def _tc_combine(h, agg, W1, b1, W2, b2, prediction_layer):
  blk = 10000
  return pl.pallas_call(
      _make_comb_body(prediction_layer),
      grid=(N // blk,),
      in_specs=[
          pl.BlockSpec((blk, F), lambda i: (i, 0)),
          pl.BlockSpec((blk, F), lambda i: (i, 0)),
          pl.BlockSpec((2 * F, 4 * F), lambda i: (0, 0)),
          pl.BlockSpec((1, 4 * F), lambda i: (0, 0)),
          pl.BlockSpec((4 * F, F), lambda i: (0, 0)),
          pl.BlockSpec((1, F), lambda i: (0, 0)),
      ],
      out_specs=pl.BlockSpec((blk, F), lambda i: (i, 0)),
      out_shape=jax.ShapeDtypeStruct((N, F), jnp.float32),
  )(h, agg, W1, b1.reshape(1, 4 * F), W2, b2.reshape(1, F))


def kernel(content, node_ids, edge_index, node_emb, proj_W, proj_b,
           c0W1, c0b1, c0W2, c0b2, c1W1, c1b1, c1W2, c1b2):
  h0 = _tc_init(content, proj_W, proj_b) + lax.slice(node_emb, (1, 0),
                                                     (N + 1, F))

  eflat = edge_index.reshape(2 * E)

  agg0, w = _sc_scatter(h0, eflat, compute_w=True)
  rw = 1.0 / jnp.maximum(w - 1.0, 1.0)
  a0 = (agg0 - h0) * rw[:, None]
  h1 = _tc_combine(h0, a0, c0W1, c0b1, c0W2, c0b2, prediction_layer=False)
  agg1 = _sc_scatter(h1, eflat, compute_w=False)
  a1 = (agg1 - h1) * rw[:, None]
  h2 = _tc_combine(h1, a1, c1W1, c1b1, c1W2, c1b2, prediction_layer=True)
  return h2


# speedup vs baseline: 1.2356x; 1.2356x over previous
import functools

import jax
import jax.numpy as jnp
from jax import lax
from jax.experimental import pallas as pl
from jax.experimental.pallas import tpu as pltpu
from jax.experimental.pallas import tpu_sc as plsc

N = 100000
E = 1600000
F = 32
DC = 128

NC = 2
NS = 16

RN = N // NC
DUMP0 = 50048
NDUMP = 64
RPAD = 50176
ZROWS = 784

K = 160
NCH = 625
EPT = NCH * K


def _sc_scatter(h, eflat, compute_w):
  mesh = plsc.VectorSubcoreMesh(
      core_axis_name="c", subcore_axis_name="s", num_cores=NC,
      num_subcores=NS)
  if compute_w:
    out_type = [jax.ShapeDtypeStruct((N, F), jnp.float32),
                jax.ShapeDtypeStruct((N,), jnp.float32)]
  else:
    out_type = jax.ShapeDtypeStruct((N, F), jnp.float32)

  scratch = (
      [pltpu.VMEM((K,), jnp.int32) for _ in range(4)]
      + [pltpu.VMEM((K,), jnp.int32) for _ in range(4)]
      + [pltpu.VMEM((K,), jnp.int32) for _ in range(4)]
      + [pltpu.VMEM((K, F), jnp.float32) for _ in range(4)]
      + [pltpu.VMEM((K,), jnp.float32)]
      + ([pltpu.VMEM((3136,), jnp.float32)] if compute_w else [])
      + [pltpu.VMEM_SHARED((RPAD, F), jnp.float32)]
      + ([pltpu.VMEM_SHARED((RPAD,), jnp.float32)] if compute_w else [])
      + [pltpu.SemaphoreType.DMA] * 12
  )

  def body(h_hbm, e_hbm, *refs):
    if compute_w:
      agg_out, w_out = refs[0], refs[1]
      refs = refs[2:]
    else:
      agg_out = refs[0]
      refs = refs[1:]
    sidx = refs[0:4]
    didx = refs[4:8]
    lix = refs[8:12]
    rws = refs[12:16]
    onesb = refs[16]
    if compute_w:
      zb1 = refs[17]
      agg_sh, w_sh = refs[18], refs[19]
      sems = refs[20:]
    else:
      agg_sh = refs[17]
      w_sh = None
      sems = refs[18:]
    isem = sems[0:4]
    gsem = sems[4:8]
    ssem = sems[8:12]

    c = lax.axis_index("c")
    s = lax.axis_index("s")
    b0 = c * RN

    zv = jnp.zeros((16,), jnp.float32)

    def zfill(i, _):
      rws[0][i, pl.ds(0, 16)] = zv
      rws[0][i, pl.ds(16, 16)] = zv
      return _
    lax.fori_loop(0, K, zfill, None)

    ov = jnp.ones((16,), jnp.float32)
    for j in range(K // 16):
      onesb[pl.ds(j * 16, 16)] = ov

    for q in range(3136 // K):
      pltpu.sync_copy(rws[0], agg_sh.at[pl.ds(s * 3136 + q * K, K)])
    zrem = 3136 % K
    if zrem:
      pltpu.sync_copy(rws[0].at[pl.ds(0, zrem)],
                      agg_sh.at[pl.ds(s * 3136 + 3136 - zrem, zrem)])
    if compute_w:
      def zfill1(i, _):
        zb1[pl.ds(i * 16, 16)] = zv
        return _
      lax.fori_loop(0, 3136 // 16, zfill1, None)
      pltpu.sync_copy(zb1, w_sh.at[pl.ds(s * 3136, 3136)])
    plsc.subcore_barrier()

    def eslice(t, which):
      off = pl.multiple_of(which * E + (s * NCH + t) * K, K)
      return e_hbm.at[pl.ds(off, K)]

    def idload_fire(t, x):
      pltpu.async_copy(eslice(t, 0), sidx[x], isem[x])
      pltpu.async_copy(eslice(t, 1), didx[x], isem[x])

    def idload_wait(t, x):
      pltpu.make_async_copy(eslice(t, 0), sidx[x], isem[x]).wait()
      pltpu.make_async_copy(eslice(t, 1), didx[x], isem[x]).wait()

    def gather_fire(x):
      pltpu.async_copy(h_hbm.at[sidx[x]], rws[x], gsem[x])

    def gather_drain(x):
      pltpu.make_async_copy(h_hbm.at[sidx[x]], rws[x], gsem[x]).wait()

    def scatter_fire(x):
      pltpu.async_copy(rws[x], agg_sh.at[lix[x]], ssem[x], add=True)
      if compute_w:
        pltpu.async_copy(onesb, w_sh.at[lix[x]], ssem[x], add=True)

    def scatter_drain(x):
      pltpu.make_async_copy(rws[x], agg_sh.at[lix[x]], ssem[x]).wait()
      if compute_w:
        pltpu.make_async_copy(onesb, w_sh.at[lix[x]], ssem[x]).wait()

    def mkidx(x):
      for ii in range(K // 16):
        d = didx[x][pl.ds(ii * 16, 16)]
        inr = (d >= b0) & (d < b0 + RN)
        loc = d - b0
        dmp = DUMP0 + (d & (NDUMP - 1))
        lix[x][pl.ds(ii * 16, 16)] = jnp.where(inr, loc, dmp)

    idload_fire(0, 0)
    idload_fire(1, 1)

    def iter_body(i, _):
      t0 = 4 * i
      for j in range(4):
        t = t0 + j
        x = j
        o = (j + 2) % 4

        @pl.when(t >= 4)
        def _():
          scatter_drain(x)

        idload_wait(t, x)
        gather_fire(x)
        mkidx(x)

        @pl.when(t >= 2)
        def _():
          gather_drain(o)
          scatter_fire(o)

        @pl.when(t < NCH - 2)
        def _():
          idload_fire(t + 2, o)
      return _
    lax.fori_loop(0, NCH // 4, iter_body, None)

    scatter_drain(0)
    idload_wait(NCH - 1, 0)
    gather_fire(0)
    mkidx(0)
    gather_drain(2)
    scatter_fire(2)
    gather_drain(3)
    scatter_fire(3)
    gather_drain(0)
    scatter_fire(0)
    scatter_drain(1)
    scatter_drain(2)
    scatter_drain(3)
    scatter_drain(0)
    plsc.subcore_barrier()

    obase = c * RN

    @pl.when(s < NS - 1)
    def _():
      pltpu.sync_copy(agg_sh.at[pl.ds(s * 3128, 3128)],
                      agg_out.at[pl.ds(obase + s * 3128, 3128)])
      if compute_w:
        pltpu.sync_copy(w_sh.at[pl.ds(s * 3128, 3128)],
                        w_out.at[pl.ds(obase + s * 3128, 3128)])

    @pl.when(s == NS - 1)
    def _():
      pltpu.sync_copy(agg_sh.at[pl.ds(46920, 3080)],
                      agg_out.at[pl.ds(obase + 46920, 3080)])
      if compute_w:
        pltpu.sync_copy(w_sh.at[pl.ds(46920, 3080)],
                        w_out.at[pl.ds(obase + 46920, 3080)])

  fn = pl.kernel(
      body, out_type=out_type, mesh=mesh, scratch_types=scratch,
      compiler_params=pltpu.CompilerParams(use_tc_tiling_on_sc=False))
  return fn(h, eflat)


def _lrelu(x):
  return jnp.where(x >= 0, x, 0.1 * x)


def _init_body(cont_ref, emb_ref, pw_ref, pb_ref, out_ref):
  x = jnp.dot(cont_ref[...], pw_ref[...],
              preferred_element_type=jnp.float32) + pb_ref[...]
  out_ref[...] = emb_ref[...] + _lrelu(x)


def _tc_init(content, emb1, proj_W, proj_b):
  blk = 10000
  return pl.pallas_call(
      _init_body,
      grid=(N // blk,),
      in_specs=[
          pl.BlockSpec((blk, DC), lambda i: (i, 0)),
          pl.BlockSpec((blk, F), lambda i: (i, 0)),
          pl.BlockSpec((DC, F), lambda i: (0, 0)),
          pl.BlockSpec((1, F), lambda i: (0, 0)),
      ],
      out_specs=pl.BlockSpec((blk, F), lambda i: (i, 0)),
      out_shape=jax.ShapeDtypeStruct((N, F), jnp.float32),
  )(content, emb1, proj_W, proj_b.reshape(1, F))


def _make_comb_body(prediction_layer):
  def body(h_ref, a_ref, w_ref, w1h_ref, w1a_ref, b1_ref, w2_ref, b2_ref,
           e_ref, g_ref, out_ref):
    h4 = h_ref[...]
    a4 = a_ref[...]
    wb = jnp.dot(w_ref[...], e_ref[...],
                 preferred_element_type=jnp.float32)
    a4 = (a4 - h4) / jnp.maximum(wb - 1.0, 1.0)
    z = jnp.dot(h4, w1h_ref[...], preferred_element_type=jnp.float32)
    z = z + jnp.dot(a4, w1a_ref[...], preferred_element_type=jnp.float32)
    z = _lrelu(z + b1_ref[...])
    z = jnp.dot(z, w2_ref[...], preferred_element_type=jnp.float32)
    z = z + b2_ref[...]
    if not prediction_layer:
      z = _lrelu(z)
    n2 = jnp.dot(z * z, g_ref[...], preferred_element_type=jnp.float32)
    out_ref[...] = z / jnp.maximum(jnp.sqrt(n2), 1e-6)
  return body


NP4 = N // 4


def _tc_combine(h4, a4, w4, W1, b1, W2, b2, prediction_layer):
  blk = 5000
  eye4 = jnp.eye(4, dtype=jnp.float32)
  w1h = jnp.kron(eye4, W1[:F])
  w1a = jnp.kron(eye4, W1[F:])
  b1p = jnp.tile(b1, 4).reshape(1, 512)
  w2p = jnp.kron(eye4, W2)
  b2p = jnp.tile(b2, 4).reshape(1, 128)
  ep = jnp.kron(eye4, jnp.ones((1, F), jnp.float32))
  gp = jnp.kron(eye4, jnp.ones((F, F), jnp.float32))
  return pl.pallas_call(
      _make_comb_body(prediction_layer),
      grid=(NP4 // blk,),
      in_specs=[
          pl.BlockSpec((blk, 128), lambda i: (i, 0)),
          pl.BlockSpec((blk, 128), lambda i: (i, 0)),
          pl.BlockSpec((blk, 4), lambda i: (i, 0)),
          pl.BlockSpec((128, 512), lambda i: (0, 0)),
          pl.BlockSpec((128, 512), lambda i: (0, 0)),
          pl.BlockSpec((1, 512), lambda i: (0, 0)),
          pl.BlockSpec((512, 128), lambda i: (0, 0)),
          pl.BlockSpec((1, 128), lambda i: (0, 0)),
          pl.BlockSpec((4, 128), lambda i: (0, 0)),
          pl.BlockSpec((128, 128), lambda i: (0, 0)),
      ],
      out_specs=pl.BlockSpec((blk, 128), lambda i: (i, 0)),
      out_shape=jax.ShapeDtypeStruct((NP4, 128), jnp.float32),
  )(h4, a4, w4, w1h, w1a, b1p, w2p, b2p, ep, gp)


def kernel(content, node_ids, edge_index, node_emb, proj_W, proj_b,
           c0W1, c0b1, c0W2, c0b2, c1W1, c1b1, c1W2, c1b2):
  emb1 = lax.slice(jax.lax.optimization_barrier(node_emb), (1, 0),
                   (N + 1, F))
  h0p = _tc_init(content, emb1, proj_W, proj_b).reshape(NP4, 128)

  eflat = edge_index.reshape(2 * E)

  agg0, w = _sc_scatter(h0p.reshape(N, F), eflat, compute_w=True)
  w4 = w.reshape(NP4, 4)
  h1p = _tc_combine(h0p, agg0.reshape(NP4, 128), w4, c0W1, c0b1, c0W2,
                    c0b2, prediction_layer=False)
  agg1 = _sc_scatter(h1p.reshape(N, F), eflat, compute_w=False)
  h2p = _tc_combine(h1p, agg1.reshape(NP4, 128), w4, c1W1, c1b1, c1W2,
                    c1b2, prediction_layer=True)
  return h2p.reshape(N, F)
